# Initial kernel scaffold; baseline (speedup 1.0000x reference)
import jax
import jax.numpy as jnp
from jax.experimental import pallas as pl


def kernel(x, edge_index, edge_attr, batch, x_emb1, x_emb2, edge_emb1, edge_emb2, W1, b1, W2, b2, bn_g, bn_b, Wd1, bd1, Wd2, bd2, Wp1, bp1, Wp2, bp2):
    raise NotImplementedError("write your pallas kernel here")


# speedup vs baseline: 2.3780x; 2.3780x over previous
import functools

import jax
import jax.numpy as jnp
from jax import lax
from jax.experimental import pallas as pl
from jax.experimental.pallas import tpu as pltpu
from jax.experimental.pallas import tpu_sc as plsc

N = 10000
E = 320000
EMB = 128
FEAT = 256
HID = 128
L = 5
G = 64
EPS = 1e-5

NSUB = 16
HALF = N // 2
NP = 5120
RPS = NP // NSUB
CHUNK = 128
CHS = 158
EPAD = NSUB * CHS * CHUNK
DUMMY = 2 * N

_mesh = plsc.VectorSubcoreMesh(core_axis_name="c", subcore_axis_name="s")


def _bdot(a, b):
    return jnp.dot(a.astype(jnp.bfloat16), b.astype(jnp.bfloat16),
                   preferred_element_type=jnp.float32)


@functools.partial(
    pl.kernel,
    mesh=_mesh,
    out_type=jax.ShapeDtypeStruct((2, NP, EMB), jnp.float32),
    scratch_types=[
        pltpu.VMEM((CHS, CHUNK), jnp.int32),
        pltpu.VMEM((CHS, CHUNK), jnp.int32),
        pltpu.VMEM((CHUNK, EMB), jnp.float32),
        pltpu.VMEM((CHUNK, EMB), jnp.float32),
        pltpu.VMEM_SHARED((NP, EMB), jnp.float32),
        pltpu.SemaphoreType.DMA,
        pltpu.SemaphoreType.DMA,
    ],
)
def _scatter_sc(table, src, dst, zeros, out, srcbuf, dstbuf, rows0, rows1,
                agg, gsem0, gsem1):
    c = lax.axis_index("c")
    s = lax.axis_index("s")
    pltpu.sync_copy(zeros.at[pl.ds(s * RPS, RPS)], agg.at[pl.ds(s * RPS, RPS)])
    pltpu.sync_copy(src.at[s], srcbuf)
    pltpu.sync_copy(dst.at[c, s], dstbuf)
    plsc.subcore_barrier()

    def step(j, carry):
        j0 = 2 * j
        d0 = pltpu.async_copy(table.at[srcbuf.at[j0]], rows0, gsem0)
        d1 = pltpu.async_copy(table.at[srcbuf.at[j0 + 1]], rows1, gsem1)
        d0.wait()
        pltpu.sync_copy(rows0, agg.at[dstbuf.at[j0]], add=True)
        d1.wait()
        pltpu.sync_copy(rows1, agg.at[dstbuf.at[j0 + 1]], add=True)
        return carry

    lax.fori_loop(0, CHS // 2, step, 0)
    plsc.subcore_barrier()
    pltpu.sync_copy(agg.at[pl.ds(s * RPS, RPS)], out.at[c, pl.ds(s * RPS, RPS)])


def _embed_body(x_ref, e1_ref, e2_ref, cp_ref, h0_ref, cnt_ref):
    x0 = x_ref[:, 0:1]
    x1 = x_ref[:, 1:2]
    h0 = jnp.zeros((N, EMB), jnp.float32)
    for k in range(3):
        h0 = h0 + jnp.where(x0 == k, 1.0, 0.0) * e1_ref[k:k + 1, :]
        h0 = h0 + jnp.where(x1 == k, 1.0, 0.0) * e2_ref[k:k + 1, :]
    h0_ref[...] = h0
    cnt_ref[...] = jnp.concatenate(
        [cp_ref[0, :HALF, :16], cp_ref[1, :HALF, :16]], axis=0)


_embed_tc = pl.pallas_call(
    _embed_body,
    out_shape=(
        jax.ShapeDtypeStruct((N, EMB), jnp.float32),
        jax.ShapeDtypeStruct((N, 16), jnp.float32),
    ),
)


def _layer_body(relu_after, s_ref, h_ref, cnt_ref, et_ref, es_ref,
                w1_ref, b1_ref, w2_ref, b2_ref, g_ref, bb_ref, o_ref):
    s_full = jnp.concatenate(
        [s_ref[0, :HALF, :], s_ref[1, :HALF, :]], axis=0)
    agg = (s_full + h_ref[...] + es_ref[...]
           + jnp.dot(cnt_ref[...], et_ref[...],
                     preferred_element_type=jnp.float32,
                     precision=lax.Precision.HIGHEST))
    hmid = jnp.maximum(_bdot(agg, w1_ref[...]) + b1_ref[...], 0.0)
    h2 = _bdot(hmid, w2_ref[...]) + b2_ref[...]
    mean = jnp.mean(h2, axis=0, keepdims=True)
    ctr = h2 - mean
    var = jnp.mean(ctr * ctr, axis=0, keepdims=True)
    hn = ctr * lax.rsqrt(var + EPS) * g_ref[...] + bb_ref[...]
    if relu_after:
        hn = jnp.maximum(hn, 0.0)
    o_ref[...] = hn


_layer_tc = {
    flag: pl.pallas_call(
        functools.partial(_layer_body, flag),
        out_shape=jax.ShapeDtypeStruct((N, EMB), jnp.float32),
    )
    for flag in (True, False)
}


def _pool_body(h_ref, b_ref, wd1_ref, bd1_ref, wd2_ref, bd2_ref,
               wp1_ref, bp1_ref, wp2_ref, bp2_ref, hdec_ref, out_ref):
    gids = lax.broadcasted_iota(jnp.int32, (G, N), 0)
    mask = jnp.where(b_ref[...] == gids, 1.0, 0.0)
    hsum = jnp.dot(mask, h_ref[...], preferred_element_type=jnp.float32,
                     precision=lax.Precision.HIGHEST)
    cntg = jnp.sum(mask, axis=1, keepdims=True)
    hg = hsum / jnp.maximum(cntg, 1.0)
    hd1 = jnp.maximum(_bdot(hg, wd1_ref[...]) + bd1_ref[...], 0.0)
    hdec = _bdot(hd1, wd2_ref[...]) + bd2_ref[...]
    hdec_ref[...] = hdec
    t = _bdot(hdec, wp1_ref[...]) + bp1_ref[...]
    sp = jnp.maximum(t, 0.0) + jnp.log(1.0 + jnp.exp(-jnp.abs(t)))
    out_ref[...] = _bdot(sp, wp2_ref[...]) + bp2_ref[...]


_pool_tc = pl.pallas_call(
    _pool_body,
    out_shape=(
        jax.ShapeDtypeStruct((G, FEAT), jnp.float32),
        jax.ShapeDtypeStruct((G, 2), jnp.float32),
    ),
)


def kernel(x, edge_index, edge_attr, batch, x_emb1, x_emb2, edge_emb1,
           edge_emb2, W1, b1, W2, b2, bn_g, bn_b, Wd1, bd1, Wd2, bd2,
           Wp1, bp1, Wp2, bp2):
    pad = EPAD - E
    srcp = jnp.concatenate([edge_index[0], jnp.zeros((pad,), jnp.int32)])
    dstp = jnp.concatenate([edge_index[1], jnp.full((pad,), DUMMY, jnp.int32)])
    src_r = srcp.reshape(NSUB, CHS, CHUNK)
    lo = dstp // HALF
    dst_r = jnp.stack([jnp.where(lo == 0, dstp, HALF),
                       jnp.where(lo == 1, dstp - HALF, HALF)]
                      ).reshape(2, NSUB, CHS, CHUNK)
    a = edge_attr[:, 0] * 3 + edge_attr[:, 1]
    a_r = jnp.concatenate([a, jnp.full((pad,), 9, jnp.int32)]
                          ).reshape(NSUB, CHS, CHUNK)
    zeros = jnp.zeros((NP, EMB), jnp.float32)
    eye = jnp.eye(16, EMB, dtype=jnp.float32)

    cnt_parts = _scatter_sc(eye, a_r, dst_r, zeros)
    h, cnt = _embed_tc(x, x_emb1[:3], x_emb2, cnt_parts)

    for l in range(L):
        e1l, e2l = edge_emb1[l], edge_emb2[l]
        etab = jnp.concatenate(
            [jnp.stack([e1l[k // 3] + e2l[k % 3] for k in range(9)]),
             jnp.zeros((7, EMB), jnp.float32)])
        eself = (e1l[4] + e2l[0]).reshape(1, EMB)
        s_parts = _scatter_sc(h, src_r, dst_r, zeros)
        h = _layer_tc[l < L - 1](
            s_parts, h, cnt, etab, eself,
            W1[l], b1[l].reshape(1, -1), W2[l], b2[l].reshape(1, -1),
            bn_g[l].reshape(1, -1), bn_b[l].reshape(1, -1))

    return _pool_tc(h, batch.reshape(1, N), Wd1, bd1.reshape(1, -1),
                    Wd2, bd2.reshape(1, -1), Wp1, bp1.reshape(1, -1),
                    Wp2, bp2.reshape(1, -1))


# speedup vs baseline: 4.1787x; 1.7572x over previous
import functools

import jax
import jax.numpy as jnp
from jax import lax
from jax.experimental import pallas as pl
from jax.experimental.pallas import tpu as pltpu
from jax.experimental.pallas import tpu_sc as plsc

N = 10000
E = 320000
EMB = 128
FEAT = 256
HID = 128
L = 5
G = 64
EPS = 1e-5

NSUB = 16
HALF = N // 2
NP = 5120
RPS = NP // NSUB
CHUNK = 128
CHS = 158
EPAD = NSUB * CHS * CHUNK
DUMMY = 2 * N
REP = 625

_mesh = plsc.VectorSubcoreMesh(core_axis_name="c", subcore_axis_name="s")


def _bdot(a, b):
    return jnp.dot(a.astype(jnp.bfloat16), b.astype(jnp.bfloat16),
                   preferred_element_type=jnp.float32)


@functools.partial(
    pl.kernel,
    mesh=_mesh,
    out_type=jax.ShapeDtypeStruct((2, NP, EMB), jnp.float32),
    scratch_types=[
        pltpu.VMEM((CHS, CHUNK), jnp.int32),
        pltpu.VMEM((CHS, CHUNK), jnp.int32),
        pltpu.VMEM((CHUNK, EMB), jnp.float32),
        pltpu.VMEM((CHUNK, EMB), jnp.float32),
        pltpu.VMEM_SHARED((NP, EMB), jnp.float32),
        pltpu.SemaphoreType.DMA,
        pltpu.SemaphoreType.DMA,
    ],
)
def _scatter_sc(table, src, dst, zeros, out, srcbuf, dstbuf, rows0, rows1,
                agg, gsem0, gsem1):
    c = lax.axis_index("c")
    s = lax.axis_index("s")
    pltpu.sync_copy(zeros.at[pl.ds(s * RPS, RPS)], agg.at[pl.ds(s * RPS, RPS)])
    pltpu.sync_copy(src.at[s], srcbuf)
    pltpu.sync_copy(dst.at[c, s], dstbuf)
    plsc.subcore_barrier()

    def step(j, carry):
        j0 = 2 * j
        d0 = pltpu.async_copy(table.at[srcbuf.at[j0]], rows0, gsem0)
        d1 = pltpu.async_copy(table.at[srcbuf.at[j0 + 1]], rows1, gsem1)
        d0.wait()
        pltpu.sync_copy(rows0, agg.at[dstbuf.at[j0]], add=True)
        d1.wait()
        pltpu.sync_copy(rows1, agg.at[dstbuf.at[j0 + 1]], add=True)
        return carry

    lax.fori_loop(0, CHS // 2, step, 0)
    plsc.subcore_barrier()
    pltpu.sync_copy(agg.at[pl.ds(s * RPS, RPS)], out.at[c, pl.ds(s * RPS, RPS)])


def _embed_body(x_ref, e1_ref, e2_ref, cp_ref, h0_ref, cnt_ref):
    x0 = x_ref[:, 0:1]
    x1 = x_ref[:, 1:2]
    h0 = jnp.zeros((N, EMB), jnp.float32)
    for k in range(3):
        h0 = h0 + jnp.where(x0 == k, 1.0, 0.0) * e1_ref[k:k + 1, :]
        h0 = h0 + jnp.where(x1 == k, 1.0, 0.0) * e2_ref[k:k + 1, :]
    h0_ref[...] = h0
    cnt_ref[...] = jnp.concatenate(
        [cp_ref[0, :HALF, :16], cp_ref[1, :HALF, :16]], axis=0)


_embed_tc = pl.pallas_call(
    _embed_body,
    out_shape=(
        jax.ShapeDtypeStruct((N, EMB), jnp.float32),
        jax.ShapeDtypeStruct((N, 16), jnp.float32),
    ),
)


def _layer_body(relu_after, s_ref, h_ref, cnt_ref, et_ref, es_ref,
                w1_ref, b1_ref, w2_ref, b2_ref, g_ref, bb_ref, o_ref):
    s_full = jnp.concatenate(
        [s_ref[0, :HALF, :], s_ref[1, :HALF, :]], axis=0)
    agg = (s_full + h_ref[...] + es_ref[...]
           + jnp.dot(cnt_ref[...], et_ref[...],
                     preferred_element_type=jnp.float32,
                     precision=lax.Precision.HIGHEST))
    hmid = jnp.maximum(_bdot(agg, w1_ref[...]) + b1_ref[...], 0.0)
    h2 = _bdot(hmid, w2_ref[...]) + b2_ref[...]
    mean = jnp.mean(h2, axis=0, keepdims=True)
    ctr = h2 - mean
    var = jnp.mean(ctr * ctr, axis=0, keepdims=True)
    hn = ctr * lax.rsqrt(var + EPS) * g_ref[...] + bb_ref[...]
    if relu_after:
        hn = jnp.maximum(hn, 0.0)
    o_ref[...] = hn


_layer_tc = {
    flag: pl.pallas_call(
        functools.partial(_layer_body, flag),
        out_shape=jax.ShapeDtypeStruct((N, EMB), jnp.float32),
    )
    for flag in (True, False)
}


def _pool_body(h_ref, b_ref, wd1_ref, bd1_ref, wd2_ref, bd2_ref,
               wp1_ref, bp1_ref, wp2_ref, bp2_ref, hdec_ref, out_ref):
    gids = lax.broadcasted_iota(jnp.int32, (G, N), 0)
    mask = jnp.where(b_ref[...] == gids, 1.0, 0.0)
    hsum = jnp.dot(mask, h_ref[...], preferred_element_type=jnp.float32,
                     precision=lax.Precision.HIGHEST)
    cntg = jnp.sum(mask, axis=1, keepdims=True)
    hg = hsum / jnp.maximum(cntg, 1.0)
    hd1 = jnp.maximum(_bdot(hg, wd1_ref[...]) + bd1_ref[...], 0.0)
    hdec = _bdot(hd1, wd2_ref[...]) + bd2_ref[...]
    hdec_ref[...] = hdec
    t = _bdot(hdec, wp1_ref[...]) + bp1_ref[...]
    sp = jnp.maximum(t, 0.0) + jnp.log(1.0 + jnp.exp(-jnp.abs(t)))
    out_ref[...] = _bdot(sp, wp2_ref[...]) + bp2_ref[...]


_pool_tc = pl.pallas_call(
    _pool_body,
    out_shape=(
        jax.ShapeDtypeStruct((G, FEAT), jnp.float32),
        jax.ShapeDtypeStruct((G, 2), jnp.float32),
    ),
)


def kernel(x, edge_index, edge_attr, batch, x_emb1, x_emb2, edge_emb1,
           edge_emb2, W1, b1, W2, b2, bn_g, bn_b, Wd1, bd1, Wd2, bd2,
           Wp1, bp1, Wp2, bp2):
    pad = EPAD - E
    srcp = jnp.concatenate([edge_index[0], jnp.zeros((pad,), jnp.int32)])
    dstp = jnp.concatenate([edge_index[1], jnp.full((pad,), DUMMY, jnp.int32)])
    src_r = srcp.reshape(NSUB, CHS, CHUNK)
    lo = dstp // HALF
    dst_r = jnp.stack([jnp.where(lo == 0, dstp, HALF),
                       jnp.where(lo == 1, dstp - HALF, HALF)]
                      ).reshape(2, NSUB, CHS, CHUNK)
    a = edge_attr[:, 0] * 3 + edge_attr[:, 1]
    spread = (jnp.arange(EPAD, dtype=jnp.int32) % REP) * 16
    a_r = (jnp.concatenate([a, jnp.full((pad,), 9, jnp.int32)]) + spread
           ).reshape(NSUB, CHS, CHUNK)
    zeros = jnp.zeros((NP, EMB), jnp.float32)
    eye = jnp.tile(jnp.eye(16, EMB, dtype=jnp.float32), (REP, 1))

    cnt_parts = _scatter_sc(eye, a_r, dst_r, zeros)
    h, cnt = _embed_tc(x, x_emb1[:3], x_emb2, cnt_parts)

    for l in range(L):
        e1l, e2l = edge_emb1[l], edge_emb2[l]
        etab = jnp.concatenate(
            [jnp.stack([e1l[k // 3] + e2l[k % 3] for k in range(9)]),
             jnp.zeros((7, EMB), jnp.float32)])
        eself = (e1l[4] + e2l[0]).reshape(1, EMB)
        s_parts = _scatter_sc(h, src_r, dst_r, zeros)
        h = _layer_tc[l < L - 1](
            s_parts, h, cnt, etab, eself,
            W1[l], b1[l].reshape(1, -1), W2[l], b2[l].reshape(1, -1),
            bn_g[l].reshape(1, -1), bn_b[l].reshape(1, -1))

    return _pool_tc(h, batch.reshape(1, N), Wd1, bd1.reshape(1, -1),
                    Wd2, bd2.reshape(1, -1), Wp1, bp1.reshape(1, -1),
                    Wp2, bp2.reshape(1, -1))
